# baseline (speedup 1.0000x reference)
import jax
import jax.numpy as jnp
from jax import lax
from jax.experimental import pallas as pl
from jax.experimental.pallas import tpu as pltpu
from jax.experimental.pallas import tpu_sc as plsc

N = 10000
E = 320000
D = 128

NC = 2
NS = 16
NW = NC * NS
CHUNK = 80
UNIT = NW * CHUNK
BK = UNIT

SIZES = (125,)


def _proj_body(x_ref, wst_ref, wdt_ref, ps_ref, pd_ref):
    xv = x_ref[...]
    ps_ref[...] = jnp.dot(xv, wst_ref[...], preferred_element_type=jnp.float32)
    pd_ref[...] = jnp.dot(xv, wdt_ref[...], preferred_element_type=jnp.float32)


def _node_projections(x, wst, wdt):
    return pl.pallas_call(
        _proj_body,
        out_shape=(
            jax.ShapeDtypeStruct((N, D), jnp.float32),
            jax.ShapeDtypeStruct((N, D), jnp.float32),
        ),
    )(x, wst, wdt)


def _make_sc_body(epw, nchunk):
    def _sc_body(ps_hbm, pd_hbm, src_hbm, dst_hbm, out_hbm, src_v, dst_v,
                 bs0, bd0, bs1, bd1, bs2, bd2,
                 ss0, sd0, ss1, sd1, ss2, sd2, so0, so1, so2):
        wid = lax.axis_index("s") * NC + lax.axis_index("c")
        base = wid * epw
        pltpu.sync_copy(src_hbm.at[pl.ds(base, epw)], src_v)
        pltpu.sync_copy(dst_hbm.at[pl.ds(base, epw)], dst_v)

        slots = ((bs0, bd0, ss0, sd0, so0),
                 (bs1, bd1, ss1, sd1, so1),
                 (bs2, bd2, ss2, sd2, so2))

        def start(ci, sl):
            bs, bd, ss, sd, _ = sl
            off = pl.multiple_of(ci * CHUNK, CHUNK)
            pltpu.async_copy(ps_hbm.at[src_v.at[pl.ds(off, CHUNK)]], bs, ss)
            pltpu.async_copy(pd_hbm.at[dst_v.at[pl.ds(off, CHUNK)]], bd, sd)

        def wait_store(ci, sl):
            bs, _, _, _, so = sl
            off = pl.multiple_of(ci * CHUNK, CHUNK)
            pltpu.make_async_copy(bs, out_hbm.at[pl.ds(base + off, CHUNK)], so).wait()

        def process(ci, sl):
            bs, bd, ss, sd, so = sl
            off = pl.multiple_of(ci * CHUNK, CHUNK)
            pltpu.make_async_copy(ps_hbm.at[src_v.at[pl.ds(off, CHUNK)]], bs, ss).wait()
            pltpu.make_async_copy(pd_hbm.at[dst_v.at[pl.ds(off, CHUNK)]], bd, sd).wait()

            @plsc.parallel_loop(0, CHUNK, 1, unroll=8)
            def add_row(r):
                for j in range(D // 16):
                    sl2 = pl.ds(j * 16, 16)
                    plsc.addupdate(bs.at[r, sl2], bd[r, sl2])
            pltpu.async_copy(bs, out_hbm.at[pl.ds(base + off, CHUNK)], so)

        start(0, slots[0])
        start(1, slots[1])
        k3 = (nchunk - 2) // 3

        def body(g, _):
            c0 = g * 3
            for k in range(3):
                c = c0 + k
                process(c, slots[k])

                @pl.when(c > 0)
                def _(c=c, k=k):
                    wait_store(c - 1, slots[(k + 2) % 3])

                start(c + 2, slots[(k + 2) % 3])
            return 0

        lax.fori_loop(0, k3, body, 0)
        for c in range(3 * k3, nchunk):
            process(c, slots[c % 3])
            wait_store(c - 1, slots[(c - 1) % 3])
            if c + 2 < nchunk:
                start(c + 2, slots[(c + 2) % 3])
        wait_store(nchunk - 1, slots[(nchunk - 1) % 3])

    return _sc_body


def _edge_gather_add(ps, pd, srck, dstk, rows):
    epw = rows // NW
    nchunk = epw // CHUNK
    mesh = plsc.VectorSubcoreMesh(core_axis_name="c", subcore_axis_name="s")
    return pl.kernel(
        _make_sc_body(epw, nchunk),
        out_type=jax.ShapeDtypeStruct((rows, D), jnp.float32),
        mesh=mesh,
        scratch_types=(
            [pltpu.VMEM((epw,), jnp.int32)] * 2
            + [pltpu.VMEM((CHUNK, D), jnp.float32)] * 6
            + [pltpu.SemaphoreType.DMA] * 9
        ),
    )(ps, pd, srck, dstk)


def _final_body(e_ref, wet_ref, b_ref, g_ref, out_ref):
    out_ref[...] = (
        jnp.dot(e_ref[...], wet_ref[...], preferred_element_type=jnp.float32)
        + g_ref[...]
        + b_ref[...]
    )


def _final_body_aliased(e_ref, wet_ref, b_ref, g_ref, prev_ref, out_ref):
    del prev_ref
    _final_body(e_ref, wet_ref, b_ref, g_ref, out_ref)


def _final_slice(e, wet, b2, g, out_prev, base_blk, nblk):
    in_specs = [
        pl.BlockSpec((BK, 16), lambda i: (base_blk + i, 0)),
        pl.BlockSpec((16, D), lambda i: (0, 0)),
        pl.BlockSpec((1, D), lambda i: (0, 0)),
        pl.BlockSpec((BK, D), lambda i: (i, 0)),
    ]
    out_spec = pl.BlockSpec((BK, D), lambda i: (base_blk + i, 0))
    out_shape = jax.ShapeDtypeStruct((E, D), jnp.float32)
    if out_prev is None:
        return pl.pallas_call(
            _final_body,
            grid=(nblk,),
            in_specs=in_specs,
            out_specs=out_spec,
            out_shape=out_shape,
        )(e, wet, b2, g)
    return pl.pallas_call(
        _final_body_aliased,
        grid=(nblk,),
        in_specs=in_specs + [pl.BlockSpec(memory_space=pl.ANY)],
        out_specs=out_spec,
        out_shape=out_shape,
        input_output_aliases={4: 0},
    )(e, wet, b2, g, out_prev)


def kernel(x, e, edge_index, W, b):
    wet = W[:, :16].T
    wst = W[:, 16:16 + D].T
    wdt = W[:, 16 + D:].T
    src = edge_index[0]
    dst = edge_index[1]
    b2 = b.reshape(1, D)
    ps, pd = _node_projections(x, wst, wdt)

    bounds = []
    r0 = 0
    for s in SIZES:
        bounds.append((r0, s * UNIT))
        r0 += s * UNIT

    gs = [
        _edge_gather_add(ps, pd, src[r0:r0 + rows], dst[r0:r0 + rows], rows)
        for r0, rows in bounds
    ]
    out = None
    for (r0, rows), g in zip(bounds, gs):
        out = _final_slice(e, wet, b2, g, out, r0 // BK, rows // BK)
    return out


# speedup vs baseline: 1.0287x; 1.0287x over previous
import jax
import jax.numpy as jnp
from jax import lax
from jax.experimental import pallas as pl
from jax.experimental.pallas import tpu as pltpu
from jax.experimental.pallas import tpu_sc as plsc

N = 10000
E = 320000
D = 128

NC = 2
NS = 16
NW = NC * NS
CHUNK = 80
UNIT = NW * CHUNK
BK = 3200

SIZES = (125,)


def _proj_body(x_ref, wst_ref, wdt_ref, ps_ref, pd_ref):
    xv = x_ref[...]
    ps_ref[...] = jnp.dot(xv, wst_ref[...], preferred_element_type=jnp.float32)
    pd_ref[...] = jnp.dot(xv, wdt_ref[...], preferred_element_type=jnp.float32)


def _node_projections(x, wst, wdt):
    return pl.pallas_call(
        _proj_body,
        out_shape=(
            jax.ShapeDtypeStruct((N, D), jnp.float32),
            jax.ShapeDtypeStruct((N, D), jnp.float32),
        ),
    )(x, wst, wdt)


def _make_sc_body(epw, nchunk):
    def _sc_body(ps_hbm, pd_hbm, src_hbm, dst_hbm, out_hbm, src_v, dst_v,
                 bs0, bd0, bs1, bd1, bs2, bd2,
                 ss0, sd0, ss1, sd1, ss2, sd2, so0, so1, so2):
        wid = lax.axis_index("s") * NC + lax.axis_index("c")
        base = wid * epw
        pltpu.sync_copy(src_hbm.at[pl.ds(base, epw)], src_v)
        pltpu.sync_copy(dst_hbm.at[pl.ds(base, epw)], dst_v)

        slots = ((bs0, bd0, ss0, sd0, so0),
                 (bs1, bd1, ss1, sd1, so1),
                 (bs2, bd2, ss2, sd2, so2))

        def start(ci, sl):
            bs, bd, ss, sd, _ = sl
            off = pl.multiple_of(ci * CHUNK, CHUNK)
            pltpu.async_copy(ps_hbm.at[src_v.at[pl.ds(off, CHUNK)]], bs, ss)
            pltpu.async_copy(pd_hbm.at[dst_v.at[pl.ds(off, CHUNK)]], bd, sd)

        def wait_store(ci, sl):
            bs, _, _, _, so = sl
            off = pl.multiple_of(ci * CHUNK, CHUNK)
            pltpu.make_async_copy(bs, out_hbm.at[pl.ds(base + off, CHUNK)], so).wait()

        def process(ci, sl):
            bs, bd, ss, sd, so = sl
            off = pl.multiple_of(ci * CHUNK, CHUNK)
            pltpu.make_async_copy(ps_hbm.at[src_v.at[pl.ds(off, CHUNK)]], bs, ss).wait()
            pltpu.make_async_copy(pd_hbm.at[dst_v.at[pl.ds(off, CHUNK)]], bd, sd).wait()

            @plsc.parallel_loop(0, CHUNK, 1, unroll=8)
            def add_row(r):
                for j in range(D // 16):
                    sl2 = pl.ds(j * 16, 16)
                    plsc.addupdate(bs.at[r, sl2], bd[r, sl2])
            pltpu.async_copy(bs, out_hbm.at[pl.ds(base + off, CHUNK)], so)

        start(0, slots[0])
        start(1, slots[1])
        k3 = (nchunk - 2) // 3

        def body(g, _):
            c0 = g * 3
            for k in range(3):
                c = c0 + k
                process(c, slots[k])

                @pl.when(c > 0)
                def _(c=c, k=k):
                    wait_store(c - 1, slots[(k + 2) % 3])

                start(c + 2, slots[(k + 2) % 3])
            return 0

        lax.fori_loop(0, k3, body, 0)
        for c in range(3 * k3, nchunk):
            process(c, slots[c % 3])
            wait_store(c - 1, slots[(c - 1) % 3])
            if c + 2 < nchunk:
                start(c + 2, slots[(c + 2) % 3])
        wait_store(nchunk - 1, slots[(nchunk - 1) % 3])

    return _sc_body


def _edge_gather_add(ps, pd, srck, dstk, rows):
    epw = rows // NW
    nchunk = epw // CHUNK
    mesh = plsc.VectorSubcoreMesh(core_axis_name="c", subcore_axis_name="s")
    return pl.kernel(
        _make_sc_body(epw, nchunk),
        out_type=jax.ShapeDtypeStruct((rows, D), jnp.float32),
        mesh=mesh,
        scratch_types=(
            [pltpu.VMEM((epw,), jnp.int32)] * 2
            + [pltpu.VMEM((CHUNK, D), jnp.float32)] * 6
            + [pltpu.SemaphoreType.DMA] * 9
        ),
    )(ps, pd, srck, dstk)


def _final_body(e_ref, wet_ref, b_ref, g_ref, out_ref):
    out_ref[...] = (
        jnp.dot(e_ref[...], wet_ref[...], preferred_element_type=jnp.float32)
        + g_ref[...]
        + b_ref[...]
    )


def _final_body_aliased(e_ref, wet_ref, b_ref, g_ref, prev_ref, out_ref):
    del prev_ref
    _final_body(e_ref, wet_ref, b_ref, g_ref, out_ref)


def _final_slice(e, wet, b2, g, out_prev, base_blk, nblk):
    in_specs = [
        pl.BlockSpec((BK, 16), lambda i: (base_blk + i, 0)),
        pl.BlockSpec((16, D), lambda i: (0, 0)),
        pl.BlockSpec((1, D), lambda i: (0, 0)),
        pl.BlockSpec((BK, D), lambda i: (i, 0)),
    ]
    out_spec = pl.BlockSpec((BK, D), lambda i: (base_blk + i, 0))
    out_shape = jax.ShapeDtypeStruct((E, D), jnp.float32)
    if out_prev is None:
        return pl.pallas_call(
            _final_body,
            grid=(nblk,),
            in_specs=in_specs,
            out_specs=out_spec,
            out_shape=out_shape,
        )(e, wet, b2, g)
    return pl.pallas_call(
        _final_body_aliased,
        grid=(nblk,),
        in_specs=in_specs + [pl.BlockSpec(memory_space=pl.ANY)],
        out_specs=out_spec,
        out_shape=out_shape,
        input_output_aliases={4: 0},
    )(e, wet, b2, g, out_prev)


def kernel(x, e, edge_index, W, b):
    wet = W[:, :16].T
    wst = W[:, 16:16 + D].T
    wdt = W[:, 16 + D:].T
    src = edge_index[0]
    dst = edge_index[1]
    b2 = b.reshape(1, D)
    ps, pd = _node_projections(x, wst, wdt)

    bounds = []
    r0 = 0
    for s in SIZES:
        bounds.append((r0, s * UNIT))
        r0 += s * UNIT

    gs = [
        _edge_gather_add(ps, pd, src[r0:r0 + rows], dst[r0:r0 + rows], rows)
        for r0, rows in bounds
    ]
    out = None
    for (r0, rows), g in zip(bounds, gs):
        out = _final_slice(e, wet, b2, g, out, r0 // BK, rows // BK)
    return out
